# Initial kernel scaffold; baseline (speedup 1.0000x reference)
import jax
import jax.numpy as jnp
from jax.experimental import pallas as pl


def kernel(x, adjacency, boundary_down, boundary_up, W, b, attention):
    raise NotImplementedError("write your pallas kernel here")


# speedup vs baseline: 1.7624x; 1.7624x over previous
import jax
import jax.numpy as jnp
from jax.experimental import pallas as pl
from jax.experimental.pallas import tpu as pltpu

N = 4096
D = 256
BM = 256
GRID = N // BM

_DN_NT = (((1,), (0,)), ((), ()))
_DN_TN = (((0,), (0,)), ((), ()))
_DN_NN = (((1,), (1,)), ((), ()))

_BF = jnp.bfloat16
_F32 = jnp.float32


def _kern(a_ref, bd_ref, bu_ref, x_ref, w_ref, b_ref, att_ref, out_ref,
          comb_ref, accd_ref, accu_ref, xtb_ref):
    i = pl.program_id(0)

    @pl.when(i == 0)
    def _init():
        xt = jax.lax.dot_general(
            x_ref[...].astype(_BF), w_ref[...].astype(_BF), _DN_NN,
            preferred_element_type=_F32) + b_ref[...]
        xtb_ref[...] = xt.astype(_BF)
        accd_ref[...] = jnp.zeros((N, D), _F32)
        accu_ref[...] = jnp.zeros((N, D), _F32)

    xtb = xtb_ref[...]
    ab = a_ref[...].astype(_BF)
    comb_ref[pl.ds(i * BM, BM), :] = jax.lax.dot_general(
        ab, xtb, _DN_NT, preferred_element_type=_F32)
    bdb = bd_ref[...].astype(_BF)
    ldn = jax.lax.dot_general(bdb, xtb, _DN_NT, preferred_element_type=_F32)
    accd_ref[...] += jax.lax.dot_general(
        bdb, ldn.astype(_BF), _DN_TN, preferred_element_type=_F32)
    bub = bu_ref[...].astype(_BF)
    lup = jax.lax.dot_general(bub, xtb, _DN_NT, preferred_element_type=_F32)
    accu_ref[...] += jax.lax.dot_general(
        bub, lup.astype(_BF), _DN_TN, preferred_element_type=_F32)

    @pl.when(i == GRID - 1)
    def _final():
        comb = comb_ref[...] + 0.5 * (accd_ref[...] + accu_ref[...])
        combb = comb.astype(_BF).astype(_F32)
        attb = att_ref[...].astype(_BF).astype(_F32)
        logit = jnp.sum(combb * attb, axis=1, keepdims=True)
        out_ref[...] = jax.nn.sigmoid(logit) * comb


def kernel(x, adjacency, boundary_down, boundary_up, W, b, attention):
    b2 = b.reshape(1, D)
    return pl.pallas_call(
        _kern,
        grid=(GRID,),
        in_specs=[
            pl.BlockSpec((BM, N), lambda i: (i, 0)),
            pl.BlockSpec((BM, N), lambda i: (i, 0)),
            pl.BlockSpec((BM, N), lambda i: (i, 0)),
            pl.BlockSpec((N, D), lambda i: (0, 0)),
            pl.BlockSpec((D, D), lambda i: (0, 0)),
            pl.BlockSpec((1, D), lambda i: (0, 0)),
            pl.BlockSpec((1, D), lambda i: (0, 0)),
        ],
        out_specs=pl.BlockSpec((N, D), lambda i: (0, 0)),
        out_shape=jax.ShapeDtypeStruct((N, D), _F32),
        scratch_shapes=[
            pltpu.VMEM((N, D), _F32),
            pltpu.VMEM((N, D), _F32),
            pltpu.VMEM((N, D), _F32),
            pltpu.VMEM((N, D), _BF),
        ],
    )(adjacency, boundary_down, boundary_up, x, W, b2, attention)
